# baseline (speedup 1.0000x reference)
import functools
import jax
import jax.numpy as jnp
from jax import lax
from jax.experimental import pallas as pl
from jax.experimental.pallas import tpu as pltpu
from jax.experimental.pallas import tpu_sc as plsc

NF = 128
H = 200
W = 200
NW = 32
NROWS = (H + NW - 1) // NW


def _sc_pos_enc(col_hbm, rowflat_hbm, out_hbm, bufa_v, bufb_v, rowvec_v,
                sem_a, sem_b):
    wid = lax.axis_index("s") * 2 + lax.axis_index("c")

    pltpu.sync_copy(col_hbm, bufa_v.at[:, pl.ds(0, NF)])
    pltpu.sync_copy(col_hbm, bufb_v.at[:, pl.ds(0, NF)])

    bufs = (bufa_v, bufb_v)
    sems = (sem_a, sem_b)

    def row_of(t):
        return jnp.minimum(t * NW + wid, H - 1)

    def fill_row_half(buf, i):
        pltpu.sync_copy(rowflat_hbm.at[pl.ds(i * NF, NF)], rowvec_v)
        chunks = [rowvec_v[pl.ds(q * 16, 16)] for q in range(NF // 16)]

        def body(r, inner):
            for q in range(NF // 16):
                buf[r, pl.ds(NF + q * 16, 16)] = chunks[q]
            return inner

        lax.fori_loop(0, W, body, 0)

    for t in range(NROWS):
        buf, sem = bufs[t % 2], sems[t % 2]
        if t >= 2:
            iprev = row_of(t - 2)
            pltpu.make_async_copy(
                buf, out_hbm.at[pl.ds(iprev * W, W)], sem).wait()
        i = row_of(t)
        fill_row_half(buf, i)
        pltpu.async_copy(buf, out_hbm.at[pl.ds(i * W, W)], sem)

    for t in (NROWS - 2, NROWS - 1):
        i = row_of(t)
        pltpu.make_async_copy(
            bufs[t % 2], out_hbm.at[pl.ds(i * W, W)], sems[t % 2]).wait()


def kernel(bev_mask, row_embed, col_embed):
    b = bev_mask.shape[0]
    h, w = bev_mask.shape[-2], bev_mask.shape[-1]
    nf = row_embed.shape[1]

    mesh = plsc.VectorSubcoreMesh(core_axis_name="c", subcore_axis_name="s")
    run = functools.partial(
        pl.kernel,
        mesh=mesh,
        out_type=jax.ShapeDtypeStruct((h * w, 2 * nf), jnp.float32),
        scratch_types=[
            pltpu.VMEM((w, 2 * nf), jnp.float32),
            pltpu.VMEM((w, 2 * nf), jnp.float32),
            pltpu.VMEM((nf,), jnp.float32),
            pltpu.SemaphoreType.DMA,
            pltpu.SemaphoreType.DMA,
        ],
    )(_sc_pos_enc)
    pos2d = run(col_embed[:w], row_embed[:h].reshape(-1))
    out = jnp.transpose(pos2d.reshape(h, w, 2 * nf), (2, 0, 1))[None]
    return jnp.broadcast_to(out, (b, 2 * nf, h, w))


# speedup vs baseline: 3.2282x; 3.2282x over previous
import jax
import jax.numpy as jnp
from jax.experimental import pallas as pl

IB = 24


def _pos_enc_kernel(row_ref, col_ref, out_ref, *, w, nf):
    ce = col_ref[...]
    out_ref[:, :, 0:nf] = jnp.broadcast_to(ce[None], (IB, w, nf))
    rv = row_ref[...]
    out_ref[:, :, nf:2 * nf] = jnp.broadcast_to(rv[:, None, :], (IB, w, nf))


def kernel(bev_mask, row_embed, col_embed):
    b = bev_mask.shape[0]
    h, w = bev_mask.shape[-2], bev_mask.shape[-1]
    nf = row_embed.shape[1]

    import functools
    body = functools.partial(_pos_enc_kernel, w=w, nf=nf)

    grid = (h + IB - 1) // IB
    pos = pl.pallas_call(
        body,
        grid=(grid,),
        in_specs=[
            pl.BlockSpec((IB, nf), lambda i: (i, 0)),
            pl.BlockSpec((w, nf), lambda i: (0, 0)),
        ],
        out_specs=pl.BlockSpec((IB, w, 2 * nf), lambda i: (i, 0, 0)),
        out_shape=jax.ShapeDtypeStruct((h, w, 2 * nf), jnp.float32),
    )(row_embed[:h], col_embed[:w])
    out = jnp.transpose(pos, (2, 0, 1))[None]
    return jnp.broadcast_to(out, (b, 2 * nf, h, w))
